# Initial kernel scaffold; baseline (speedup 1.0000x reference)
import jax
import jax.numpy as jnp
from jax.experimental import pallas as pl


def kernel(x, edge_index, batch, W1, b1, W2, b2, W_ih, W_hh, b_ih, b_hh, Wfc, bfc):
    raise NotImplementedError("write your pallas kernel here")


# speedup vs baseline: 39.1144x; 39.1144x over previous
import functools

import jax
import jax.numpy as jnp
from jax import lax
from jax.experimental import pallas as pl
from jax.experimental.pallas import tpu as pltpu
from jax.experimental.pallas import tpu_sc as plsc

N = 10000
E = 320000
D_IN = 128
H = 16
D_OUT = 128
B = 100
T = 100

NC = 2
NS = 16
EPC = E // NC
EPT = EPC // NS
CH = 2000
NCHUNK = EPT // CH
SUB = 16
SUBLEN = CH // SUB
NP = 10240
ROWS_PT = NP // NS
DEG_PAD = 10240
DEG_PT = DEG_PAD // NS

@functools.cache
def _mesh():
    return plsc.VectorSubcoreMesh(core_axis_name="c", subcore_axis_name="s",
                                  num_cores=NC, num_subcores=NS)


def _sc_deg_body(dst_hbm, out_hbm, acc, dstbuf, onesbuf, zbuf, sems):
    cid = lax.axis_index("c")
    sid = lax.axis_index("s")

    def _zero(i, _):
        zbuf[pl.ds(i * 16, 16)] = jnp.zeros((16,), jnp.float32)
        onesbuf[pl.ds(i * 16, 16)] = jnp.ones((16,), jnp.float32)
        return 0
    lax.fori_loop(0, DEG_PT // 16, _zero, 0, unroll=True)
    pltpu.sync_copy(zbuf, acc.at[pl.ds(sid * DEG_PT, DEG_PT)])
    plsc.subcore_barrier()

    base = (cid * NS + sid) * EPT

    def _chunk(k, _):
        off = base + k * CH
        roff = pl.multiple_of(off // SUBLEN, SUB)
        pltpu.sync_copy(dst_hbm.at[pl.ds(roff, SUB)], dstbuf)
        descs = []
        for j in range(SUB):
            descs.append(pltpu.async_copy(
                onesbuf.at[pl.ds(0, SUBLEN)],
                acc.at[dstbuf.at[j]], sems, add=True))
        for d in descs:
            d.wait()
        return 0
    lax.fori_loop(0, NCHUNK, _chunk, 0)

    plsc.subcore_barrier()
    pltpu.sync_copy(acc.at[pl.ds(sid * DEG_PT, DEG_PT)], zbuf)
    pltpu.sync_copy(zbuf, out_hbm.at[cid, pl.ds(sid * DEG_PT, DEG_PT)])


@functools.cache
def _sc_deg_kernel():
    return pl.kernel(
        _sc_deg_body,
        out_type=jax.ShapeDtypeStruct((NC, DEG_PAD), jnp.float32),
        mesh=_mesh(),
        compiler_params=pltpu.CompilerParams(use_tc_tiling_on_sc=False),
        scratch_types=[
            pltpu.VMEM_SHARED((DEG_PAD,), jnp.float32),
            pltpu.VMEM((SUB, SUBLEN), jnp.int32),
            pltpu.VMEM((DEG_PT,), jnp.float32),
            pltpu.VMEM((DEG_PT,), jnp.float32),
            pltpu.SemaphoreType.DMA,
        ],
    )


def _sc_deg(dst2):
    return _sc_deg_kernel()(dst2)


def _sc_agg_body(g_hbm, src_hbm, dst_hbm, out_hbm, acc, srcbuf, dstbuf, rows,
                 zbuf, semg, sems):
    cid = lax.axis_index("c")
    sid = lax.axis_index("s")

    def _zero(i, _):
        zbuf[i] = jnp.zeros((16,), jnp.float32)
        return 0
    lax.fori_loop(0, ROWS_PT, _zero, 0)
    pltpu.sync_copy(zbuf, acc.at[pl.ds(sid * ROWS_PT, ROWS_PT)])
    plsc.subcore_barrier()

    base = (cid * NS + sid) * EPT

    def _chunk(k, _):
        off = pl.multiple_of(base + k * CH, CH)
        roff = pl.multiple_of(off // SUBLEN, SUB)
        pltpu.sync_copy(src_hbm.at[pl.ds(off, CH)], srcbuf)
        pltpu.sync_copy(dst_hbm.at[pl.ds(roff, SUB)], dstbuf)
        pltpu.async_copy(g_hbm.at[srcbuf], rows, semg).wait()
        descs = []
        for j in range(SUB):
            descs.append(pltpu.async_copy(
                rows.at[pl.ds(j * SUBLEN, SUBLEN)],
                acc.at[dstbuf.at[j]], sems, add=True))
        for d in descs:
            d.wait()
        return 0
    lax.fori_loop(0, NCHUNK, _chunk, 0)

    plsc.subcore_barrier()
    pltpu.sync_copy(acc.at[pl.ds(sid * ROWS_PT, ROWS_PT)], zbuf)
    pltpu.sync_copy(zbuf, out_hbm.at[cid, pl.ds(sid * ROWS_PT, ROWS_PT)])


@functools.cache
def _sc_agg_kernel():
    return pl.kernel(
        _sc_agg_body,
        out_type=jax.ShapeDtypeStruct((NC, NP, H), jnp.float32),
        mesh=_mesh(),
        compiler_params=pltpu.CompilerParams(use_tc_tiling_on_sc=False),
        scratch_types=[
            pltpu.VMEM_SHARED((NP, H), jnp.float32),
            pltpu.VMEM((CH,), jnp.int32),
            pltpu.VMEM((SUB, SUBLEN), jnp.int32),
            pltpu.VMEM((CH, H), jnp.float32),
            pltpu.VMEM((ROWS_PT, H), jnp.float32),
            pltpu.SemaphoreType.DMA,
            pltpu.SemaphoreType.DMA,
        ],
    )


def _sc_agg(g, src, dst2):
    return _sc_agg_kernel()(g, src, dst2)


RB = 1000


def _tc_g1_body(x_ref, d0_ref, d1_ref, w1_ref, out_ref, dinv_ref):
    dinv = lax.rsqrt(d0_ref[...] + d1_ref[...] + 1.0)
    h = jnp.dot(x_ref[...], w1_ref[...], preferred_element_type=jnp.float32)
    out_ref[...] = dinv * h
    dinv_ref[...] = dinv


def _tc_g1(x, d0, d1, W1):
    return pl.pallas_call(
        _tc_g1_body,
        grid=(N // RB,),
        in_specs=[
            pl.BlockSpec((RB, D_IN), lambda i: (i, 0)),
            pl.BlockSpec((RB, 1), lambda i: (i, 0)),
            pl.BlockSpec((RB, 1), lambda i: (i, 0)),
            pl.BlockSpec((D_IN, H), lambda i: (0, 0)),
        ],
        out_specs=[
            pl.BlockSpec((RB, H), lambda i: (i, 0)),
            pl.BlockSpec((RB, 1), lambda i: (i, 0)),
        ],
        out_shape=[
            jax.ShapeDtypeStruct((N, H), jnp.float32),
            jax.ShapeDtypeStruct((N, 1), jnp.float32),
        ],
    )(x, d0, d1, W1)


def _tc_g2_body(s0_ref, s1_ref, g1_ref, dinv_ref, w2_ref, b1_ref, out_ref):
    dinv = dinv_ref[...]
    h1 = jax.nn.relu(dinv * (s0_ref[...] + s1_ref[...] + g1_ref[...])
                     + b1_ref[...])
    out_ref[...] = dinv * jnp.dot(h1, w2_ref[...],
                                  preferred_element_type=jnp.float32)


def _tc_g2(s0, s1, g1, dinv, W2, b1):
    return pl.pallas_call(
        _tc_g2_body,
        grid=(N // RB,),
        in_specs=[
            pl.BlockSpec((RB, H), lambda i: (i, 0)),
            pl.BlockSpec((RB, H), lambda i: (i, 0)),
            pl.BlockSpec((RB, H), lambda i: (i, 0)),
            pl.BlockSpec((RB, 1), lambda i: (i, 0)),
            pl.BlockSpec((H, H), lambda i: (0, 0)),
            pl.BlockSpec((1, H), lambda i: (0, 0)),
        ],
        out_specs=pl.BlockSpec((RB, H), lambda i: (i, 0)),
        out_shape=jax.ShapeDtypeStruct((N, H), jnp.float32),
    )(s0, s1, g1, dinv, W2, b1)


def _tc_gru_body(s0_ref, s1_ref, g2_ref, dinv_ref, b2_ref,
                 wir_ref, wiz_ref, win_ref, whr_ref, whz_ref, whn_ref,
                 bir_ref, biz_ref, bin_ref, bhr_ref, bhz_ref, bhn_ref,
                 wfc_ref, bfc_ref, out_ref, h_scr):
    t = pl.program_id(0)

    @pl.when(t == 0)
    def _():
        h_scr[...] = jnp.zeros((B, H), jnp.float32)

    pre = s0_ref[:, 0, 0, :] + s1_ref[:, 0, 0, :] + g2_ref[:, 0, 0, :]
    xt = jax.nn.relu(dinv_ref[:, 0, 0, :] * pre + b2_ref[...])
    h = h_scr[...]

    def mm(a, w):
        return jnp.dot(a, w[...], preferred_element_type=jnp.float32)

    r = jax.nn.sigmoid(mm(xt, wir_ref) + bir_ref[...]
                       + mm(h, whr_ref) + bhr_ref[...])
    z = jax.nn.sigmoid(mm(xt, wiz_ref) + biz_ref[...]
                       + mm(h, whz_ref) + bhz_ref[...])
    n = jnp.tanh(mm(xt, win_ref) + bin_ref[...]
                 + r * (mm(h, whn_ref) + bhn_ref[...]))
    hn = (1.0 - z) * n + z * h
    h_scr[...] = hn

    @pl.when(t == T - 1)
    def _():
        out_ref[...] = mm(hn, wfc_ref) + bfc_ref[...]


def _tc_gru(s0, s1, g2, dinv, b2, W_ih, W_hh, b_ih, b_hh, Wfc, bfc):
    s04 = s0.reshape(B, T, 1, H)
    s14 = s1.reshape(B, T, 1, H)
    g24 = g2.reshape(B, T, 1, H)
    dinv4 = dinv.reshape(B, T, 1, 1)
    tslice = pl.BlockSpec((B, 1, 1, H), lambda t: (0, t, 0, 0))
    dslice = pl.BlockSpec((B, 1, 1, 1), lambda t: (0, t, 0, 0))

    def const(shape):
        nd = len(shape)
        return pl.BlockSpec(shape, lambda t: (0,) * nd)

    wargs = [W_ih[:, :H], W_ih[:, H:2 * H], W_ih[:, 2 * H:],
             W_hh[:, :H], W_hh[:, H:2 * H], W_hh[:, 2 * H:]]
    bargs = [b_ih[None, :H], b_ih[None, H:2 * H], b_ih[None, 2 * H:],
             b_hh[None, :H], b_hh[None, H:2 * H], b_hh[None, 2 * H:]]
    return pl.pallas_call(
        _tc_gru_body,
        grid=(T,),
        in_specs=[tslice, tslice, tslice, dslice, const((1, H))]
        + [const((H, H))] * 6 + [const((1, H))] * 6
        + [const((H, D_OUT)), const((1, D_OUT))],
        out_specs=pl.BlockSpec((B, D_OUT), lambda t: (0, 0)),
        out_shape=jax.ShapeDtypeStruct((B, D_OUT), jnp.float32),
        scratch_shapes=[pltpu.VMEM((B, H), jnp.float32)],
    )(s04, s14, g24, dinv4, b2[None, :], *wargs, *bargs, Wfc, bfc[None, :])


def kernel(x, edge_index, batch, W1, b1, W2, b2, W_ih, W_hh, b_ih, b_hh,
           Wfc, bfc):
    src = edge_index[0]
    dst = edge_index[1]
    dst2 = dst.reshape(E // SUBLEN, SUBLEN)

    degp = _sc_deg(dst2)
    d0 = degp[0, :N].reshape(N, 1)
    d1 = degp[1, :N].reshape(N, 1)

    g1, dinv = _tc_g1(x, d0, d1, W1)
    s = _sc_agg(g1, src, dst2)[:, :N]
    g2 = _tc_g2(s[0], s[1], g1, dinv, W2, b1[None, :])
    s2 = _sc_agg(g2, src, dst2)[:, :N]
    return _tc_gru(s2[0], s2[1], g2, dinv, b2, W_ih, W_hh, b_ih, b_hh,
                   Wfc, bfc)
